# baseline (speedup 1.0000x reference)
import jax
import jax.numpy as jnp
from jax.experimental import pallas as pl
from jax.experimental.pallas import tpu as pltpu

SEQ_BLK = 256


def _pe_add_kernel(x_ref, pe_ref, o_ref, *, batch):
    pe = pe_ref[...]
    pe_b = jnp.concatenate([pe] * batch, axis=1)
    o_ref[...] = x_ref[...] + pe_b


def kernel(x, pe_weight):
    seq_len, batch, d_model = x.shape
    x2 = x.reshape(seq_len, batch * d_model)
    grid = (seq_len // SEQ_BLK,)
    import functools
    out2 = pl.pallas_call(
        functools.partial(_pe_add_kernel, batch=batch),
        grid=grid,
        in_specs=[
            pl.BlockSpec((SEQ_BLK, batch * d_model), lambda i: (i, 0)),
            pl.BlockSpec((SEQ_BLK, d_model), lambda i: (i, 0)),
        ],
        out_specs=pl.BlockSpec((SEQ_BLK, batch * d_model), lambda i: (i, 0)),
        out_shape=jax.ShapeDtypeStruct((seq_len, batch * d_model), x.dtype),
        compiler_params=pltpu.CompilerParams(
            dimension_semantics=("parallel",),
        ),
    )(x2, pe_weight)
    return out2.reshape(seq_len, batch, d_model)


# speedup vs baseline: 3.8514x; 3.8514x over previous
import jax
import jax.numpy as jnp
from jax.experimental import pallas as pl
from jax.experimental.pallas import tpu as pltpu

SEQ_BLK = 512


def _pe_add_kernel(x_ref, pe_ref, o_ref):
    o_ref[...] = x_ref[...] + pe_ref[...][:, None, :]


def kernel(x, pe_weight):
    seq_len, batch, d_model = x.shape
    grid = (seq_len // SEQ_BLK,)
    return pl.pallas_call(
        _pe_add_kernel,
        grid=grid,
        in_specs=[
            pl.BlockSpec((SEQ_BLK, batch, d_model), lambda i: (i, 0, 0)),
            pl.BlockSpec((SEQ_BLK, d_model), lambda i: (i, 0)),
        ],
        out_specs=pl.BlockSpec((SEQ_BLK, batch, d_model), lambda i: (i, 0, 0)),
        out_shape=jax.ShapeDtypeStruct((seq_len, batch, d_model), x.dtype),
        compiler_params=pltpu.CompilerParams(
            dimension_semantics=("parallel",),
        ),
    )(x, pe_weight)
